# baseline (speedup 1.0000x reference)
import functools

import jax
import jax.numpy as jnp
from jax import lax
from jax.experimental import pallas as pl
from jax.experimental.pallas import tpu as pltpu
from jax.experimental.pallas import tpu_sc as plsc

VOCAB = 1000000
HALF = 32
N_EMBD = 64

NC = 2
NS = 16
NW = NC * NS
B, T = 4096, 50
N = B * T
BW = B // NW
NCHUNK = T
NB = 5
NGROUP = NCHUNK // NB
OUT_CH = BW * N_EMBD


def _body(idx_hbm, real_hbm, imag_hbm, out_hbm,
          idx_v, real_v, imag_v, out_v, sem_r, sem_i, sem_o):
    wid = lax.axis_index("c") * NS + lax.axis_index("s")
    pltpu.sync_copy(idx_hbm.at[:, wid], idx_v)

    lanes = lax.iota(jnp.int32, 16)
    q = lanes >> 2
    tl = (2 * lanes) & 7

    def fire(ci, b):
        pltpu.async_copy(real_hbm.at[idx_v.at[ci]], real_v.at[b], sem_r)
        pltpu.async_copy(imag_hbm.at[idx_v.at[ci]], imag_v.at[b], sem_i)

    for b in range(NB - 1):
        fire(b, b)

    def group_body(g, _):
        ci0 = g * NB
        for b in range(NB):
            ci = ci0 + b
            pltpu.make_async_copy(
                real_hbm.at[idx_v.at[ci]], real_v.at[b], sem_r).wait()
            pltpu.make_async_copy(
                imag_hbm.at[idx_v.at[ci]], imag_v.at[b], sem_i).wait()

            @pl.when(ci >= NB)
            def _():
                pltpu.make_async_copy(
                    out_v.at[b], out_hbm.at[ci - NB, :, wid], sem_o).wait()

            @plsc.parallel_loop(0, BW, step=1, unroll=8)
            def _row(r):
                rr = 0 * lanes + r
                plsc.store_scatter(
                    out_v.at[b], [q, tl, rr], real_v[b, r, pl.ds(0, 16)])
                plsc.store_scatter(
                    out_v.at[b], [q, tl + 1, rr], imag_v[b, r, pl.ds(0, 16)])
                plsc.store_scatter(
                    out_v.at[b], [q + 4, tl, rr], real_v[b, r, pl.ds(16, 16)])
                plsc.store_scatter(
                    out_v.at[b], [q + 4, tl + 1, rr],
                    imag_v[b, r, pl.ds(16, 16)])

            @pl.when(ci + NB - 1 < NCHUNK)
            def _():
                fire(ci + NB - 1, (b + NB - 1) % NB)

            pltpu.async_copy(
                out_v.at[b], out_hbm.at[ci, :, wid], sem_o)
        return 0

    lax.fori_loop(0, NGROUP, group_body, 0)

    for b in range(NB):
        ci = NCHUNK - NB + b
        pltpu.make_async_copy(
            out_v.at[b], out_hbm.at[ci, :, wid], sem_o).wait()


@jax.jit
def _spinor_embed(idx3, embed_real, embed_imag):
    mesh = plsc.VectorSubcoreMesh(core_axis_name="c", subcore_axis_name="s")
    run = pl.kernel(
        _body,
        out_type=jax.ShapeDtypeStruct((T, 8, NW, 8, 128), jnp.float32),
        mesh=mesh,
        scratch_types=[
            pltpu.VMEM((NCHUNK, BW), jnp.int32),
            pltpu.VMEM((NB, BW, HALF), jnp.float32),
            pltpu.VMEM((NB, BW, HALF), jnp.float32),
            pltpu.VMEM((NB, 8, 8, BW), jnp.float32),
            pltpu.SemaphoreType.DMA,
            pltpu.SemaphoreType.DMA,
            pltpu.SemaphoreType.DMA,
        ],
        compiler_params=pltpu.CompilerParams(
            needs_layout_passes=False, use_tc_tiling_on_sc=False),
    )
    return run(idx3, embed_real, embed_imag)


def kernel(idx, embed_real, embed_imag):
    idx3 = idx.T.astype(jnp.int32).reshape(T, NW, BW)
    out5 = _spinor_embed(idx3, embed_real, embed_imag)
    return out5.transpose(2, 4, 0, 1, 3).reshape(B, T, N_EMBD)


# speedup vs baseline: 1.0106x; 1.0106x over previous
import functools

import jax
import jax.numpy as jnp
from jax import lax
from jax.experimental import pallas as pl
from jax.experimental.pallas import tpu as pltpu
from jax.experimental.pallas import tpu_sc as plsc

VOCAB = 1000000
HALF = 32
N_EMBD = 64

NC = 2
NS = 16
NW = NC * NS
B, T = 4096, 50
N = B * T
PER_W = N // NW
CHUNK = 128
NCHUNK = PER_W // CHUNK
NB = 5
NGROUP = NCHUNK // NB
OUT_CH = CHUNK * N_EMBD


def _body(idx_hbm, real_hbm, imag_hbm, out_hbm,
          idx_v, real_v, imag_v, out_v, sem_r, sem_i, sem_o):
    wid = lax.axis_index("c") * NS + lax.axis_index("s")
    pltpu.sync_copy(idx_hbm.at[wid], idx_v)

    lanes = lax.iota(jnp.int32, 16)
    cols_r0 = 2 * lanes
    cols_i0 = cols_r0 + 1
    cols_r1 = cols_r0 + 32
    cols_i1 = cols_r0 + 33

    def fire(ci, b):
        pltpu.async_copy(real_hbm.at[idx_v.at[ci]], real_v.at[b], sem_r)
        pltpu.async_copy(imag_hbm.at[idx_v.at[ci]], imag_v.at[b], sem_i)

    for b in range(NB - 1):
        fire(b, b)

    def group_body(g, _):
        ci0 = g * NB
        for b in range(NB):
            ci = ci0 + b
            pltpu.make_async_copy(
                real_hbm.at[idx_v.at[ci]], real_v.at[b], sem_r).wait()
            pltpu.make_async_copy(
                imag_hbm.at[idx_v.at[ci]], imag_v.at[b], sem_i).wait()

            @pl.when(ci >= NB)
            def _():
                pltpu.make_async_copy(
                    out_v.at[b],
                    out_hbm.at[wid, pl.ds((ci - NB) * OUT_CH, OUT_CH)],
                    sem_o).wait()

            @plsc.parallel_loop(0, CHUNK, step=1, unroll=8)
            def _row(r):
                base = r * N_EMBD
                plsc.store_scatter(
                    out_v.at[b], [base + cols_r0], real_v[b, r, pl.ds(0, 16)])
                plsc.store_scatter(
                    out_v.at[b], [base + cols_i0], imag_v[b, r, pl.ds(0, 16)])
                plsc.store_scatter(
                    out_v.at[b], [base + cols_r1], real_v[b, r, pl.ds(16, 16)])
                plsc.store_scatter(
                    out_v.at[b], [base + cols_i1], imag_v[b, r, pl.ds(16, 16)])

            @pl.when(ci + NB - 1 < NCHUNK)
            def _():
                fire(ci + NB - 1, (b + NB - 1) % NB)

            pltpu.async_copy(
                out_v.at[b],
                out_hbm.at[wid, pl.ds(ci * OUT_CH, OUT_CH)], sem_o)
        return 0

    lax.fori_loop(0, NGROUP, group_body, 0)

    for b in range(NB):
        ci = NCHUNK - NB + b
        pltpu.make_async_copy(
            out_v.at[b],
            out_hbm.at[wid, pl.ds(ci * OUT_CH, OUT_CH)], sem_o).wait()


@jax.jit
def _spinor_embed(idx3, embed_real, embed_imag):
    mesh = plsc.VectorSubcoreMesh(core_axis_name="c", subcore_axis_name="s")
    run = pl.kernel(
        _body,
        out_type=jax.ShapeDtypeStruct((NW, PER_W * N_EMBD), jnp.float32),
        mesh=mesh,
        scratch_types=[
            pltpu.VMEM((NCHUNK, CHUNK), jnp.int32),
            pltpu.VMEM((NB, CHUNK, HALF), jnp.float32),
            pltpu.VMEM((NB, CHUNK, HALF), jnp.float32),
            pltpu.VMEM((NB, OUT_CH), jnp.float32),
            pltpu.SemaphoreType.DMA,
            pltpu.SemaphoreType.DMA,
            pltpu.SemaphoreType.DMA,
        ],
        compiler_params=pltpu.CompilerParams(
            needs_layout_passes=False, use_tc_tiling_on_sc=False),
    )
    return run(idx3, embed_real, embed_imag)


def kernel(idx, embed_real, embed_imag):
    idx3 = idx.astype(jnp.int32).reshape(NW, NCHUNK, CHUNK)
    out = _spinor_embed(idx3, embed_real, embed_imag)
    return out.reshape(B, T, N_EMBD)
